# baseline (speedup 1.0000x reference)
import functools

import jax
import jax.numpy as jnp
from jax import lax
from jax.experimental import pallas as pl
from jax.experimental.pallas import tpu as pltpu
from jax.experimental.pallas import tpu_sc as plsc

DIM = 64
SCALE = 8.0

_NC = 2
_NS = 16
_NW = _NC * _NS

_C = 128


@functools.partial(jax.jit, static_argnums=(2,))
def _emb_call(x_flat, lut, B):
    bpw = B // _NW
    nchunk = bpw // _C
    mesh = plsc.VectorSubcoreMesh(core_axis_name="c", subcore_axis_name="s")

    @functools.partial(
        pl.kernel,
        out_type=jax.ShapeDtypeStruct((B, DIM), jnp.float32),
        mesh=mesh,
        scratch_types=[
            pltpu.VMEM((_C,), jnp.int32),
            pltpu.VMEM((_C, DIM), jnp.float32),
            pltpu.SemaphoreType.DMA,
        ],
        compiler_params=pltpu.CompilerParams(use_tc_tiling_on_sc=False),
    )
    def emb(x_hbm, lut_hbm, out_hbm, idx_v, rows_v, sem):
        wid = lax.axis_index("s") * _NC + lax.axis_index("c")
        base = wid * bpw

        def chunk_body(g, carry):
            off = base + g * _C
            pltpu.sync_copy(x_hbm.at[pl.ds(off, _C)], idx_v)
            pltpu.async_copy(lut_hbm.at[idx_v], rows_v, sem).wait()

            def row_body(r, c2):
                for k in range(DIM // 16):
                    sl = pl.ds(k * 16, 16)
                    rows_v[r, sl] = rows_v[r, sl] * SCALE
                return c2

            lax.fori_loop(0, _C, row_body, 0)
            pltpu.sync_copy(rows_v, out_hbm.at[pl.ds(off, _C)])
            return carry

        lax.fori_loop(0, nchunk, chunk_body, 0)

    return emb(x_flat, lut)


def kernel(X, lut):
    s0, s1 = X.shape
    B = s0 * s1
    x_flat = X.reshape(B).astype(jnp.int32)
    out = _emb_call(x_flat, lut, B)
    return out.reshape(s0, s1, DIM)


# speedup vs baseline: 1.2670x; 1.2670x over previous
import functools

import jax
import jax.numpy as jnp
from jax import lax
from jax.experimental import pallas as pl
from jax.experimental.pallas import tpu as pltpu
from jax.experimental.pallas import tpu_sc as plsc

DIM = 64
SCALE = 8.0

_NC = 2
_NS = 16
_NW = _NC * _NS

_C = 128
_NBUF = 4


@functools.partial(jax.jit, static_argnums=(2,))
def _emb_call(x_grp, lut, B):
    bpw = B // _NW
    nchunk = bpw // _C
    assert nchunk % _NBUF == 0
    mesh = plsc.VectorSubcoreMesh(core_axis_name="c", subcore_axis_name="s")

    @functools.partial(
        pl.kernel,
        out_type=jax.ShapeDtypeStruct((B, DIM), jnp.float32),
        mesh=mesh,
        scratch_types=[
            pltpu.VMEM((nchunk, _C), jnp.int32),
            pltpu.VMEM((_NBUF, _C, DIM), jnp.float32),
            pltpu.SemaphoreType.DMA((_NBUF,)),
            pltpu.SemaphoreType.DMA((_NBUF,)),
        ],
        compiler_params=pltpu.CompilerParams(use_tc_tiling_on_sc=False),
    )
    def emb(x_hbm, lut_hbm, out_hbm, idx_all, rows_v, gsem, osem):
        wid = lax.axis_index("s") * _NC + lax.axis_index("c")
        base = wid * bpw
        pltpu.sync_copy(x_hbm.at[wid], idx_all)

        def gather(n, b):
            return pltpu.make_async_copy(
                lut_hbm.at[idx_all.at[n]], rows_v.at[b], gsem.at[b])

        def writeback(n, b):
            return pltpu.make_async_copy(
                rows_v.at[b], out_hbm.at[pl.ds(base + n * _C, _C)], osem.at[b])

        for b in range(_NBUF - 1):
            gather(b, b).start()

        @pl.loop(0, nchunk, step=_NBUF)
        def outer(g0):
            for b in range(_NBUF):
                g = g0 + b
                nxt = g + _NBUF - 1
                nb = (b + _NBUF - 1) % _NBUF

                @pl.when(nxt < nchunk)
                def _():
                    @pl.when(nxt >= _NBUF)
                    def _():
                        writeback(nxt - _NBUF, nb).wait()
                    gather(nxt, nb).start()

                gather(g, b).wait()

                @plsc.parallel_loop(0, _C, unroll=4)
                def scale_row(r):
                    for k in range(DIM // 16):
                        sl = pl.ds(k * 16, 16)
                        rows_v[b, r, sl] = rows_v[b, r, sl] * SCALE

                writeback(g, b).start()

        for b in range(_NBUF):
            writeback(nchunk - _NBUF + b, b).wait()

    return emb(x_grp, lut)


def kernel(X, lut):
    s0, s1 = X.shape
    B = s0 * s1
    bpw = B // _NW
    x_grp = X.reshape(_NW, bpw // _C, _C).astype(jnp.int32)
    out = _emb_call(x_grp, lut, B)
    return out.reshape(s0, s1, DIM)
